# baseline (speedup 1.0000x reference)
import functools

import jax
import jax.numpy as jnp
from jax import lax
from jax.experimental import pallas as pl
from jax.experimental.pallas import tpu as pltpu
from jax.experimental.pallas import tpu_sc as plsc

VOCAB = 100000
DIM = 128
BATCH = 1024

_NUM_CORES = 2
_NUM_SUBCORES = 16
_NUM_WORKERS = _NUM_CORES * _NUM_SUBCORES
_ROWS_PER_WORKER = BATCH // _NUM_WORKERS

_VBLK = 4992
_NBLK = (VOCAB + _VBLK - 1) // _VBLK


def _sc_gather_body(table_hbm, idx_hbm, out_hbm, idx_v, rows_v, sem):
    wid = lax.axis_index("s") * _NUM_CORES + lax.axis_index("c")
    base = wid * _ROWS_PER_WORKER
    pltpu.sync_copy(idx_hbm.at[pl.ds(base, _ROWS_PER_WORKER)], idx_v)
    pltpu.async_copy(table_hbm.at[idx_v], rows_v, sem).wait()
    pltpu.sync_copy(rows_v, out_hbm.at[pl.ds(base, _ROWS_PER_WORKER)])


@functools.cache
def _make_sc_gather():
    return pl.kernel(
        _sc_gather_body,
        mesh=plsc.VectorSubcoreMesh(
            core_axis_name="c",
            subcore_axis_name="s",
            num_cores=_NUM_CORES,
            num_subcores=_NUM_SUBCORES,
        ),
        out_type=jax.ShapeDtypeStruct((BATCH, DIM), jnp.float32),
        scratch_types=[
            pltpu.VMEM((_ROWS_PER_WORKER,), jnp.int32),
            pltpu.VMEM((_ROWS_PER_WORKER, DIM), jnp.float32),
            pltpu.SemaphoreType.DMA,
        ],
    )


def _mlp_body(x_ref, w1_ref, b1_ref, w2t_ref, b2_ref, out_ref, h_ref):
    @pl.when(pl.program_id(0) == 0)
    def _():
        h = jnp.dot(x_ref[...], w1_ref[...], preferred_element_type=jnp.float32)
        h_ref[...] = jnp.maximum(h + b1_ref[...], 0.0)

    out = lax.dot_general(
        w2t_ref[...],
        h_ref[...],
        (((1,), (1,)), ((), ())),
        preferred_element_type=jnp.float32,
    )
    out_ref[...] = out + jnp.reshape(b2_ref[...], (_VBLK, 1))


def kernel(tokens, embed_table, W1, b1, W2, b2):
    x = _make_sc_gather()(embed_table, tokens)

    w2t = jnp.swapaxes(W2, 0, 1)
    out_t = pl.pallas_call(
        _mlp_body,
        grid=(_NBLK,),
        in_specs=[
            pl.BlockSpec((BATCH, DIM), lambda i: (0, 0)),
            pl.BlockSpec((DIM, DIM), lambda i: (0, 0)),
            pl.BlockSpec((1, DIM), lambda i: (0, 0)),
            pl.BlockSpec((_VBLK, DIM), lambda i: (i, 0)),
            pl.BlockSpec((1, _VBLK), lambda i: (0, i)),
        ],
        out_specs=pl.BlockSpec((_VBLK, BATCH), lambda i: (i, 0)),
        out_shape=jax.ShapeDtypeStruct((VOCAB, BATCH), jnp.float32),
        scratch_shapes=[pltpu.VMEM((BATCH, DIM), jnp.float32)],
    )(x, W1, b1.reshape(1, DIM), w2t, b2.reshape(1, VOCAB))
    return jnp.swapaxes(out_t, 0, 1)


# speedup vs baseline: 1.0046x; 1.0046x over previous
import functools

import jax
import jax.numpy as jnp
from jax import lax
from jax.experimental import pallas as pl
from jax.experimental.pallas import tpu as pltpu
from jax.experimental.pallas import tpu_sc as plsc

VOCAB = 100000
DIM = 128
BATCH = 1024

_NUM_CORES = 2
_NUM_SUBCORES = 16
_NUM_WORKERS = _NUM_CORES * _NUM_SUBCORES
_ROWS_PER_WORKER = BATCH // _NUM_WORKERS

_VBLK = 6144
_NBLK = (VOCAB + _VBLK - 1) // _VBLK


def _sc_gather_body(table_hbm, idx_hbm, out_hbm, idx_v, rows_v, sem):
    wid = lax.axis_index("s") * _NUM_CORES + lax.axis_index("c")
    base = wid * _ROWS_PER_WORKER
    pltpu.sync_copy(idx_hbm.at[pl.ds(base, _ROWS_PER_WORKER)], idx_v)
    pltpu.async_copy(table_hbm.at[idx_v], rows_v, sem).wait()
    pltpu.sync_copy(rows_v, out_hbm.at[pl.ds(base, _ROWS_PER_WORKER)])


@functools.cache
def _make_sc_gather():
    return pl.kernel(
        _sc_gather_body,
        mesh=plsc.VectorSubcoreMesh(
            core_axis_name="c",
            subcore_axis_name="s",
            num_cores=_NUM_CORES,
            num_subcores=_NUM_SUBCORES,
        ),
        out_type=jax.ShapeDtypeStruct((BATCH, DIM), jnp.float32),
        scratch_types=[
            pltpu.VMEM((_ROWS_PER_WORKER,), jnp.int32),
            pltpu.VMEM((_ROWS_PER_WORKER, DIM), jnp.float32),
            pltpu.SemaphoreType.DMA,
        ],
    )


def _mlp_body(x_ref, w1_ref, b1_ref, w2t_ref, b2_ref, out_ref, h_ref):
    @pl.when(pl.program_id(0) == 0)
    def _():
        h = jnp.dot(x_ref[...], w1_ref[...], preferred_element_type=jnp.float32)
        h_ref[...] = jnp.maximum(h + b1_ref[...], 0.0)

    out = lax.dot_general(
        w2t_ref[...],
        h_ref[...],
        (((1,), (1,)), ((), ())),
        preferred_element_type=jnp.float32,
    )
    out_ref[...] = out + jnp.reshape(b2_ref[...], (_VBLK, 1))


def kernel(tokens, embed_table, W1, b1, W2, b2):
    x = _make_sc_gather()(embed_table, tokens)

    w2t = jnp.swapaxes(W2, 0, 1)
    out_t = pl.pallas_call(
        _mlp_body,
        grid=(_NBLK,),
        in_specs=[
            pl.BlockSpec((BATCH, DIM), lambda i: (0, 0)),
            pl.BlockSpec((DIM, DIM), lambda i: (0, 0)),
            pl.BlockSpec((1, DIM), lambda i: (0, 0)),
            pl.BlockSpec((_VBLK, DIM), lambda i: (i, 0)),
            pl.BlockSpec((1, _VBLK), lambda i: (0, i)),
        ],
        out_specs=pl.BlockSpec((_VBLK, BATCH), lambda i: (i, 0)),
        out_shape=jax.ShapeDtypeStruct((VOCAB, BATCH), jnp.float32),
        scratch_shapes=[pltpu.VMEM((BATCH, DIM), jnp.float32)],
    )(x, W1, b1.reshape(1, DIM), w2t, b2.reshape(1, VOCAB))
    return jnp.swapaxes(out_t, 0, 1)


# speedup vs baseline: 1.0095x; 1.0049x over previous
import functools

import jax
import jax.numpy as jnp
from jax import lax
from jax.experimental import pallas as pl
from jax.experimental.pallas import tpu as pltpu
from jax.experimental.pallas import tpu_sc as plsc

VOCAB = 100000
DIM = 128
BATCH = 1024

_NUM_CORES = 2
_NUM_SUBCORES = 16
_NUM_WORKERS = _NUM_CORES * _NUM_SUBCORES
_ROWS_PER_WORKER = BATCH // _NUM_WORKERS

_VBLK = 6144
_NBLK = (VOCAB + _VBLK - 1) // _VBLK


_HALF = _ROWS_PER_WORKER // 2


def _sc_gather_body(table_hbm, idx_hbm, out_hbm, idx_v, rows_v, sem_g0, sem_g1, sem_s0, sem_s1):
    wid = lax.axis_index("s") * _NUM_CORES + lax.axis_index("c")
    base = wid * _ROWS_PER_WORKER
    pltpu.sync_copy(idx_hbm.at[pl.ds(base, _ROWS_PER_WORKER)], idx_v)
    g0 = pltpu.async_copy(
        table_hbm.at[idx_v.at[pl.ds(0, _HALF)]], rows_v.at[pl.ds(0, _HALF)], sem_g0
    )
    g1 = pltpu.async_copy(
        table_hbm.at[idx_v.at[pl.ds(_HALF, _HALF)]],
        rows_v.at[pl.ds(_HALF, _HALF)],
        sem_g1,
    )
    g0.wait()
    s0 = pltpu.async_copy(
        rows_v.at[pl.ds(0, _HALF)], out_hbm.at[pl.ds(base, _HALF)], sem_s0
    )
    g1.wait()
    s1 = pltpu.async_copy(
        rows_v.at[pl.ds(_HALF, _HALF)],
        out_hbm.at[pl.ds(base + _HALF, _HALF)],
        sem_s1,
    )
    s0.wait()
    s1.wait()


@functools.cache
def _make_sc_gather():
    return pl.kernel(
        _sc_gather_body,
        mesh=plsc.VectorSubcoreMesh(
            core_axis_name="c",
            subcore_axis_name="s",
            num_cores=_NUM_CORES,
            num_subcores=_NUM_SUBCORES,
        ),
        out_type=jax.ShapeDtypeStruct((BATCH, DIM), jnp.float32),
        scratch_types=[
            pltpu.VMEM((_ROWS_PER_WORKER,), jnp.int32),
            pltpu.VMEM((_ROWS_PER_WORKER, DIM), jnp.float32),
            pltpu.SemaphoreType.DMA,
            pltpu.SemaphoreType.DMA,
            pltpu.SemaphoreType.DMA,
            pltpu.SemaphoreType.DMA,
        ],
    )


def _mlp_body(x_ref, w1_ref, b1_ref, w2t_ref, b2_ref, out_ref, h_ref):
    @pl.when(pl.program_id(0) == 0)
    def _():
        h = jnp.dot(x_ref[...], w1_ref[...], preferred_element_type=jnp.float32)
        h_ref[...] = jnp.maximum(h + b1_ref[...], 0.0)

    out = lax.dot_general(
        w2t_ref[...],
        h_ref[...],
        (((1,), (1,)), ((), ())),
        preferred_element_type=jnp.float32,
    )
    out_ref[...] = out + jnp.reshape(b2_ref[...], (_VBLK, 1))


def kernel(tokens, embed_table, W1, b1, W2, b2):
    x = _make_sc_gather()(embed_table, tokens)

    w2t = jnp.swapaxes(W2, 0, 1)
    out_t = pl.pallas_call(
        _mlp_body,
        grid=(_NBLK,),
        in_specs=[
            pl.BlockSpec((BATCH, DIM), lambda i: (0, 0)),
            pl.BlockSpec((DIM, DIM), lambda i: (0, 0)),
            pl.BlockSpec((1, DIM), lambda i: (0, 0)),
            pl.BlockSpec((_VBLK, DIM), lambda i: (i, 0)),
            pl.BlockSpec((1, _VBLK), lambda i: (0, i)),
        ],
        out_specs=pl.BlockSpec((_VBLK, BATCH), lambda i: (i, 0)),
        out_shape=jax.ShapeDtypeStruct((VOCAB, BATCH), jnp.float32),
        scratch_shapes=[pltpu.VMEM((BATCH, DIM), jnp.float32)],
    )(x, W1, b1.reshape(1, DIM), w2t, b2.reshape(1, VOCAB))
    return jnp.swapaxes(out_t, 0, 1)
